# baseline (speedup 1.0000x reference)
import functools

import jax
import jax.numpy as jnp
from jax import lax
from jax.experimental import pallas as pl
from jax.experimental.pallas import tpu as pltpu
from jax.experimental.pallas import tpu_sc as plsc

f32 = jnp.float32
NC, NS = 2, 16
CH = 128


def _degree_kernel(n_chunks, N):
  NA = N + 8
  CPT = n_chunks // NS
  assert CPT * NS == n_chunks and CPT % 8 == 0
  ZB = 2000
  K = 6
  mesh = plsc.VectorSubcoreMesh(
      core_axis_name="c", subcore_axis_name="s", num_cores=NC, num_subcores=NS)

  @functools.partial(
      pl.kernel,
      out_type=(jax.ShapeDtypeStruct((NA,), f32),
                jax.ShapeDtypeStruct((NA,), f32)),
      mesh=mesh,
      scratch_types=[
          pltpu.VMEM((CPT, CH), jnp.int32),
          pltpu.VMEM((CH,), f32),
          pltpu.VMEM((ZB,), f32),
          pltpu.VMEM_SHARED((NA,), f32),
          pltpu.SemaphoreType.DMA,
      ],
  )
  def deg_kernel(ei, deg_src_out, deg_dst_out, ibuf, ones_v, zbuf, deg_sh,
                 sem):
    c = lax.axis_index("c")
    s = lax.axis_index("s")

    def fill_ones(i, _):
      ones_v[pl.ds(i * 16, 16)] = jnp.ones((16,), f32)
      return 0
    lax.fori_loop(0, CH // 16, fill_ones, 0)

    pltpu.sync_copy(ei.at[c, pl.ds(s * CPT, CPT)], ibuf)

    @pl.when(s == 0)
    def _():
      def fz(i, _):
        zbuf[pl.ds(i * 16, 16)] = jnp.zeros((16,), f32)
        return 0
      lax.fori_loop(0, ZB // 16, fz, 0)

      def zc(i, _):
        pltpu.sync_copy(zbuf, deg_sh.at[pl.ds(i * ZB, ZB)])
        return 0
      lax.fori_loop(0, N // ZB, zc, 0)
      pltpu.sync_copy(zbuf.at[pl.ds(0, 8)], deg_sh.at[pl.ds(N, 8)])

    plsc.subcore_barrier()

    def issue(j):
      pltpu.async_copy(ones_v, deg_sh.at[ibuf.at[j]], sem, add=True)

    def wait(j):
      pltpu.make_async_copy(ones_v, deg_sh.at[ibuf.at[j]], sem).wait()

    for j in range(K):
      issue(j)

    def body(j, _):
      wait(j - K)
      issue(j)
      return 0
    lax.fori_loop(K, CPT, body, 0)

    def drain(j, _):
      wait(j)
      return 0
    lax.fori_loop(CPT - K, CPT, drain, 0)

    plsc.subcore_barrier()

    @pl.when((s == 0) & (c == 0))
    def _():
      pltpu.sync_copy(deg_sh, deg_src_out)

    @pl.when((s == 0) & (c == 1))
    def _():
      pltpu.sync_copy(deg_sh, deg_dst_out)

  return deg_kernel


def _edge_agg_kernel(n_chunks, N, D, split=0.5):
  NW = NC * NS
  CPW0 = int(round(n_chunks * split / NS))
  CPW1 = n_chunks // NS - CPW0
  assert (CPW0 + CPW1) * NS == n_chunks
  NA = N + 8
  NPT = N // NS
  ZR = 125
  mesh = plsc.VectorSubcoreMesh(
      core_axis_name="c", subcore_axis_name="s", num_cores=NC, num_subcores=NS)

  RM = (N // NS) & ~7
  REM = N - NS * RM

  assert CPW0 >= 14 and CPW1 >= 14

  @functools.partial(
      pl.kernel,
      out_type=(jax.ShapeDtypeStruct((N, D), f32),
                jax.ShapeDtypeStruct((N, D), f32)),
      mesh=mesh,
      scratch_types=[
          pltpu.VMEM((6, 2, CH), jnp.int32),
          pltpu.VMEM((CH, D), f32),
          pltpu.VMEM((CH, D), f32),
          pltpu.VMEM((CH, D), f32),
          pltpu.VMEM_SHARED((NA, D), f32),
          pltpu.SemaphoreType.DMA,
          pltpu.SemaphoreType.DMA,
          pltpu.SemaphoreType.DMA,
          pltpu.SemaphoreType.DMA,
          pltpu.SemaphoreType.DMA,
          pltpu.SemaphoreType.DMA,
          pltpu.SemaphoreType.DMA,
      ],
  )
  def agg_kernel(ei, h, agg0_out, agg1_out, islots, rows0, rows1, rows2,
                 agg_sh, sem_i0, sem_i1, sem_i2, sem_g0, sem_g1, sem_s0,
                 sem_s1):
    c = lax.axis_index("c")
    s = lax.axis_index("s")
    rows = (rows0, rows1, rows2)
    sem_i = (sem_i0, sem_i1, sem_i2)
    sem_g = (sem_g0, sem_g1)
    sem_s = (sem_s0, sem_s1)

    lanes_per_row = D // 16

    def fz(i, _):
      r = i // lanes_per_row
      col = (i % lanes_per_row) * 16
      rows0[r, pl.ds(col, 16)] = jnp.zeros((16,), f32)
      return 0
    lax.fori_loop(0, CH * lanes_per_row, fz, 0)

    def zc(i, _):
      pltpu.sync_copy(rows0.at[pl.ds(0, ZR)],
                      agg_sh.at[pl.ds(s * NPT + i * ZR, ZR)])
      return 0
    lax.fori_loop(0, NPT // ZR, zc, 0)

    plsc.subcore_barrier()

    def run_pipeline(cpw, base):
      def ii(j, sl6):
        pltpu.async_copy(ei.at[base + j], islots.at[sl6], sem_i[sl6 % 3])

      def iw(j, sl6):
        pltpu.make_async_copy(ei.at[base + j], islots.at[sl6],
                              sem_i[sl6 % 3]).wait()

      def gi(j, sl6):
        pltpu.async_copy(h.at[islots.at[sl6, 0]], rows[sl6 % 3],
                         sem_g[sl6 % 2])

      def gw(j, sl6):
        pltpu.make_async_copy(h.at[islots.at[sl6, 0]], rows[sl6 % 3],
                              sem_g[sl6 % 2]).wait()

      def si(j, sl6):
        pltpu.async_copy(rows[sl6 % 3], agg_sh.at[islots.at[sl6, 1]],
                         sem_s[sl6 % 2], add=True)

      def sw(j, sl6):
        pltpu.make_async_copy(rows[sl6 % 3], agg_sh.at[islots.at[sl6, 1]],
                              sem_s[sl6 % 2]).wait()

      def emit_body(j, u):
        static = isinstance(j, int)
        gw(j, u % 6)
        si(j, u % 6)
        if (not static) or j >= 1:
          sw(j - 1, (u - 1) % 6)
        if (not static) or j + 3 < cpw:
          ii(j + 3, (u + 3) % 6)
        if (not static) or j + 2 < cpw:
          iw(j + 2, (u + 2) % 6)
          gi(j + 2, (u + 2) % 6)

      ii(0, 0)
      ii(1, 1)
      ii(2, 2)
      iw(0, 0)
      gi(0, 0)
      iw(1, 1)
      gi(1, 1)

      gw(0, 0)
      si(0, 0)
      ii(3, 3)
      iw(2, 2)
      gi(2, 2)
      for j in range(1, 6):
        emit_body(j, j)

      n_mid = (cpw - 14) // 6

      def body(i, _):
        for u in range(6):
          emit_body(6 * i + u, u)
        return 0
      lax.fori_loop(1, 1 + n_mid, body, 0)

      for j in range(6 + 6 * n_mid, cpw):
        emit_body(j, j)
      sw(cpw - 1, (cpw - 1) % 6)

    if CPW0 == CPW1:
      run_pipeline(CPW0, (s * NC + c) * CPW0)
    else:
      @pl.when(c == 0)
      def _():
        run_pipeline(CPW0, s * CPW0)

      @pl.when(c == 1)
      def _():
        run_pipeline(CPW1, NS * CPW0 + s * CPW1)

    plsc.subcore_barrier()

    @pl.when(c == 0)
    def _():
      pltpu.sync_copy(agg_sh.at[pl.ds(s * RM, RM)],
                      agg0_out.at[pl.ds(s * RM, RM)])

      @pl.when(s == NS - 1)
      def _():
        pltpu.sync_copy(agg_sh.at[pl.ds(NS * RM, REM)],
                        agg0_out.at[pl.ds(NS * RM, REM)])

    @pl.when(c == 1)
    def _():
      pltpu.sync_copy(agg_sh.at[pl.ds(s * RM, RM)],
                      agg1_out.at[pl.ds(s * RM, RM)])

      @pl.when(s == NS - 1)
      def _():
        pltpu.sync_copy(agg_sh.at[pl.ds(NS * RM, REM)],
                        agg1_out.at[pl.ds(NS * RM, REM)])

  return agg_kernel


def _dense1_body(xt_ref, xv_ref, wt, bt, wv, bv, wp1, bp1, g_ln, be_ln,
                 wp2, bp2, wct, bct, wcv, bcv,
                 h_ref, lut_ref, luv_ref):
  xt = xt_ref[...]
  xv = xv_ref[...]
  ht = jnp.maximum(jnp.dot(xt, wt[...], preferred_element_type=f32) + bt[...],
                   0.0)
  hv = jnp.maximum(jnp.dot(xv, wv[...], preferred_element_type=f32) + bv[...],
                   0.0)
  d = wt.shape[1]
  h1 = (jnp.dot(ht, wp1[0:d, :], preferred_element_type=f32) +
        jnp.dot(hv, wp1[d:2 * d, :], preferred_element_type=f32) + bp1[...])
  h1 = jnp.maximum(h1, 0.0)
  mu = jnp.mean(h1, axis=-1, keepdims=True)
  xc = h1 - mu
  var = jnp.mean(xc * xc, axis=-1, keepdims=True)
  hn = xc * lax.rsqrt(var + 1e-5) * g_ln[...] + be_ln[...]
  h_ref[...] = jnp.dot(hn, wp2[...], preferred_element_type=f32) + bp2[...]
  lut_ref[...] = jnp.dot(ht, wct[...], preferred_element_type=f32) + bct[...]
  luv_ref[...] = jnp.dot(hv, wcv[...], preferred_element_type=f32) + bcv[...]


def _prescale_body(h_ref, degs_ref, out_ref):
  dinv = lax.rsqrt(jnp.maximum(degs_ref[0, 0, :], 1.0))
  out_ref[...] = h_ref[...] * dinv[:, None]


def _dense2_body(agg0_ref, agg1_ref, degd_ref, wg, bg, wcc, bcc, lut_ref,
                 luv_ref, lf_ref, lc_ref):
  agg = agg0_ref[...] + agg1_ref[...]
  dinv = lax.rsqrt(jnp.maximum(degd_ref[0, 0, :], 1.0))
  hc = (jnp.dot(agg * dinv[:, None], wg[...], preferred_element_type=f32)
        + bg[...])
  lc = jnp.dot(hc, wcc[...], preferred_element_type=f32) + bcc[...]
  lc_ref[...] = lc
  lf_ref[...] = lc + lut_ref[...] + luv_ref[...]


def kernel(x_t, x_v, edge_index, W_t, b_t, W_v, b_v, W_p1, b_p1, g_ln, be_ln,
           W_p2, b_p2, W_g, b_g, W_cC, b_cC, W_ct, b_ct, W_cv, b_cv):
  N, D_T = x_t.shape
  D = W_t.shape[1]
  C = W_cC.shape[1]
  E = edge_index.shape[1]
  BN = 5000
  NB = N // BN

  NW = NC * NS
  n_chunks_pad = ((E // CH + NW * 8 - 1) // (NW * 8)) * NW * 8
  pad_n = n_chunks_pad * CH - E
  if pad_n:
    pad_idx = N + (jnp.arange(pad_n, dtype=jnp.int32) % 8)
    ei_pad = jnp.concatenate(
        [edge_index, jnp.stack([pad_idx, pad_idx])], axis=1)
  else:
    ei_pad = edge_index
  ei3_pad = ei_pad.reshape(2, n_chunks_pad, CH)
  ei_t = jnp.transpose(ei3_pad, (1, 0, 2))

  deg_s, deg_d = _degree_kernel(n_chunks_pad, N)(ei3_pad)
  deg_src = deg_s[:N].reshape(NB, 1, BN)
  deg_dst = deg_d[:N].reshape(NB, 1, BN)

  row = lambda v: v.reshape(1, -1)
  full = lambda shape: pl.BlockSpec(shape, lambda i: tuple(0 for _ in shape))

  h2, logits_Ut, logits_Uv = pl.pallas_call(
      _dense1_body,
      grid=(NB,),
      in_specs=[
          pl.BlockSpec((BN, D_T), lambda i: (i, 0)),
          pl.BlockSpec((BN, D_T), lambda i: (i, 0)),
          full((D_T, D)), full((1, D)),
          full((D_T, D)), full((1, D)),
          full((2 * D, D)), full((1, D)),
          full((1, D)), full((1, D)),
          full((D, D)), full((1, D)),
          full((D, C)), full((1, C)),
          full((D, C)), full((1, C)),
      ],
      out_specs=[
          pl.BlockSpec((BN, D), lambda i: (i, 0)),
          pl.BlockSpec((BN, C), lambda i: (i, 0)),
          pl.BlockSpec((BN, C), lambda i: (i, 0)),
      ],
      out_shape=[
          jax.ShapeDtypeStruct((N, D), f32),
          jax.ShapeDtypeStruct((N, C), f32),
          jax.ShapeDtypeStruct((N, C), f32),
      ],
  )(x_t, x_v, W_t, row(b_t), W_v, row(b_v), W_p1, row(b_p1), row(g_ln),
    row(be_ln), W_p2, row(b_p2), W_ct, row(b_ct), W_cv, row(b_cv))

  h_scaled = pl.pallas_call(
      _prescale_body,
      grid=(NB,),
      in_specs=[
          pl.BlockSpec((BN, D), lambda i: (i, 0)),
          pl.BlockSpec((1, 1, BN), lambda i: (i, 0, 0)),
      ],
      out_specs=pl.BlockSpec((BN, D), lambda i: (i, 0)),
      out_shape=jax.ShapeDtypeStruct((N + 8, D), f32),
  )(h2, deg_src)

  agg0, agg1 = _edge_agg_kernel(n_chunks_pad, N, D, split=0.6)(ei_t, h_scaled)

  logits_final, logits_C = pl.pallas_call(
      _dense2_body,
      grid=(NB,),
      in_specs=[
          pl.BlockSpec((BN, D), lambda i: (i, 0)),
          pl.BlockSpec((BN, D), lambda i: (i, 0)),
          pl.BlockSpec((1, 1, BN), lambda i: (i, 0, 0)),
          full((D, D)), full((1, D)),
          full((D, C)), full((1, C)),
          pl.BlockSpec((BN, C), lambda i: (i, 0)),
          pl.BlockSpec((BN, C), lambda i: (i, 0)),
      ],
      out_specs=[
          pl.BlockSpec((BN, C), lambda i: (i, 0)),
          pl.BlockSpec((BN, C), lambda i: (i, 0)),
      ],
      out_shape=[
          jax.ShapeDtypeStruct((N, C), f32),
          jax.ShapeDtypeStruct((N, C), f32),
      ],
  )(agg0, agg1, deg_dst, W_g, row(b_g), W_cC, row(b_cC),
    logits_Ut, logits_Uv)

  return (logits_final, logits_C, logits_Ut, logits_Uv)


# speedup vs baseline: 1.0193x; 1.0193x over previous
import functools

import jax
import jax.numpy as jnp
from jax import lax
from jax.experimental import pallas as pl
from jax.experimental.pallas import tpu as pltpu
from jax.experimental.pallas import tpu_sc as plsc

f32 = jnp.float32
NC, NS = 2, 16
CH = 128


def _degree_kernel(n_chunks, N):
  NA = N + 8
  CPT = n_chunks // NS
  assert CPT * NS == n_chunks and CPT % 8 == 0
  ZB = 2000
  K = 6
  mesh = plsc.VectorSubcoreMesh(
      core_axis_name="c", subcore_axis_name="s", num_cores=NC, num_subcores=NS)

  @functools.partial(
      pl.kernel,
      out_type=(jax.ShapeDtypeStruct((NA,), f32),
                jax.ShapeDtypeStruct((NA,), f32)),
      mesh=mesh,
      scratch_types=[
          pltpu.VMEM((CPT, CH), jnp.int32),
          pltpu.VMEM((CH,), f32),
          pltpu.VMEM((ZB,), f32),
          pltpu.VMEM_SHARED((NA,), f32),
          pltpu.SemaphoreType.DMA,
      ],
  )
  def deg_kernel(ei, deg_src_out, deg_dst_out, ibuf, ones_v, zbuf, deg_sh,
                 sem):
    c = lax.axis_index("c")
    s = lax.axis_index("s")

    def fill_ones(i, _):
      ones_v[pl.ds(i * 16, 16)] = jnp.ones((16,), f32)
      return 0
    lax.fori_loop(0, CH // 16, fill_ones, 0)

    pltpu.sync_copy(ei.at[c, pl.ds(s * CPT, CPT)], ibuf)

    @pl.when(s == 0)
    def _():
      def fz(i, _):
        zbuf[pl.ds(i * 16, 16)] = jnp.zeros((16,), f32)
        return 0
      lax.fori_loop(0, ZB // 16, fz, 0)

      def zc(i, _):
        pltpu.sync_copy(zbuf, deg_sh.at[pl.ds(i * ZB, ZB)])
        return 0
      lax.fori_loop(0, N // ZB, zc, 0)
      pltpu.sync_copy(zbuf.at[pl.ds(0, 8)], deg_sh.at[pl.ds(N, 8)])

    plsc.subcore_barrier()

    def issue(j):
      pltpu.async_copy(ones_v, deg_sh.at[ibuf.at[j]], sem, add=True)

    def wait(j):
      pltpu.make_async_copy(ones_v, deg_sh.at[ibuf.at[j]], sem).wait()

    for j in range(K):
      issue(j)

    def body(j, _):
      wait(j - K)
      issue(j)
      return 0
    lax.fori_loop(K, CPT, body, 0)

    def drain(j, _):
      wait(j)
      return 0
    lax.fori_loop(CPT - K, CPT, drain, 0)

    plsc.subcore_barrier()

    @pl.when((s == 0) & (c == 0))
    def _():
      pltpu.sync_copy(deg_sh, deg_src_out)

    @pl.when((s == 0) & (c == 1))
    def _():
      pltpu.sync_copy(deg_sh, deg_dst_out)

  return deg_kernel


def _edge_agg_kernel(n_chunks, N, D, split=0.5):
  NW = NC * NS
  CPW0 = int(round(n_chunks * split / NS))
  CPW1 = n_chunks // NS - CPW0
  assert (CPW0 + CPW1) * NS == n_chunks
  NA = N + 8
  NPT = N // NS
  ZR = 125
  mesh = plsc.VectorSubcoreMesh(
      core_axis_name="c", subcore_axis_name="s", num_cores=NC, num_subcores=NS)

  RM = (N // NS) & ~7
  REM = N - NS * RM

  assert CPW0 >= 14 and CPW1 >= 14

  @functools.partial(
      pl.kernel,
      out_type=(jax.ShapeDtypeStruct((N, D), f32),
                jax.ShapeDtypeStruct((N, D), f32)),
      mesh=mesh,
      scratch_types=[
          pltpu.VMEM((6, 2, CH), jnp.int32),
          pltpu.VMEM((CH, D), f32),
          pltpu.VMEM((CH, D), f32),
          pltpu.VMEM((CH, D), f32),
          pltpu.VMEM_SHARED((NA, D), f32),
          pltpu.SemaphoreType.DMA,
          pltpu.SemaphoreType.DMA,
          pltpu.SemaphoreType.DMA,
          pltpu.SemaphoreType.DMA,
          pltpu.SemaphoreType.DMA,
          pltpu.SemaphoreType.DMA,
          pltpu.SemaphoreType.DMA,
      ],
  )
  def agg_kernel(ei, h, agg0_out, agg1_out, islots, rows0, rows1, rows2,
                 agg_sh, sem_i0, sem_i1, sem_i2, sem_g0, sem_g1, sem_s0,
                 sem_s1):
    c = lax.axis_index("c")
    s = lax.axis_index("s")
    rows = (rows0, rows1, rows2)
    sem_i = (sem_i0, sem_i1, sem_i2)
    sem_g = (sem_g0, sem_g1)
    sem_s = (sem_s0, sem_s1)

    lanes_per_row = D // 16

    def fz(i, _):
      r = i // lanes_per_row
      col = (i % lanes_per_row) * 16
      rows0[r, pl.ds(col, 16)] = jnp.zeros((16,), f32)
      return 0
    lax.fori_loop(0, CH * lanes_per_row, fz, 0)

    def zc(i, _):
      pltpu.sync_copy(rows0.at[pl.ds(0, ZR)],
                      agg_sh.at[pl.ds(s * NPT + i * ZR, ZR)])
      return 0
    lax.fori_loop(0, NPT // ZR, zc, 0)

    plsc.subcore_barrier()

    def run_pipeline(cpw, base):
      def ii(j, sl6):
        pltpu.async_copy(ei.at[base + j], islots.at[sl6], sem_i[sl6 % 3])

      def iw(j, sl6):
        pltpu.make_async_copy(ei.at[base + j], islots.at[sl6],
                              sem_i[sl6 % 3]).wait()

      def gi(j, sl6):
        pltpu.async_copy(h.at[islots.at[sl6, 0]], rows[sl6 % 3],
                         sem_g[sl6 % 2])

      def gw(j, sl6):
        pltpu.make_async_copy(h.at[islots.at[sl6, 0]], rows[sl6 % 3],
                              sem_g[sl6 % 2]).wait()

      def si(j, sl6):
        pltpu.async_copy(rows[sl6 % 3], agg_sh.at[islots.at[sl6, 1]],
                         sem_s[sl6 % 2], add=True)

      def sw(j, sl6):
        pltpu.make_async_copy(rows[sl6 % 3], agg_sh.at[islots.at[sl6, 1]],
                              sem_s[sl6 % 2]).wait()

      def emit_body(j, u):
        static = isinstance(j, int)
        gw(j, u % 6)
        si(j, u % 6)
        if (not static) or j >= 1:
          sw(j - 1, (u - 1) % 6)
        if (not static) or j + 3 < cpw:
          ii(j + 3, (u + 3) % 6)
        if (not static) or j + 2 < cpw:
          iw(j + 2, (u + 2) % 6)
          gi(j + 2, (u + 2) % 6)

      ii(0, 0)
      ii(1, 1)
      ii(2, 2)
      iw(0, 0)
      gi(0, 0)
      iw(1, 1)
      gi(1, 1)

      gw(0, 0)
      si(0, 0)
      ii(3, 3)
      iw(2, 2)
      gi(2, 2)
      for j in range(1, 6):
        emit_body(j, j)

      n_mid = (cpw - 14) // 6

      def body(i, _):
        for u in range(6):
          emit_body(6 * i + u, u)
        return 0
      lax.fori_loop(1, 1 + n_mid, body, 0)

      for j in range(6 + 6 * n_mid, cpw):
        emit_body(j, j)
      sw(cpw - 1, (cpw - 1) % 6)

    if CPW0 == CPW1:
      run_pipeline(CPW0, (s * NC + c) * CPW0)
    else:
      @pl.when(c == 0)
      def _():
        run_pipeline(CPW0, s * CPW0)

      @pl.when(c == 1)
      def _():
        run_pipeline(CPW1, NS * CPW0 + s * CPW1)

    plsc.subcore_barrier()

    @pl.when(c == 0)
    def _():
      pltpu.sync_copy(agg_sh.at[pl.ds(s * RM, RM)],
                      agg0_out.at[pl.ds(s * RM, RM)])

      @pl.when(s == NS - 1)
      def _():
        pltpu.sync_copy(agg_sh.at[pl.ds(NS * RM, REM)],
                        agg0_out.at[pl.ds(NS * RM, REM)])

    @pl.when(c == 1)
    def _():
      pltpu.sync_copy(agg_sh.at[pl.ds(s * RM, RM)],
                      agg1_out.at[pl.ds(s * RM, RM)])

      @pl.when(s == NS - 1)
      def _():
        pltpu.sync_copy(agg_sh.at[pl.ds(NS * RM, REM)],
                        agg1_out.at[pl.ds(NS * RM, REM)])

  return agg_kernel


def _dense1_body(xt_ref, xv_ref, wt, bt, wv, bv, wp1, bp1, g_ln, be_ln,
                 wp2, bp2, wct, bct, wcv, bcv,
                 h_ref, lut_ref, luv_ref):
  xt = xt_ref[...]
  xv = xv_ref[...]
  ht = jnp.maximum(jnp.dot(xt, wt[...], preferred_element_type=f32) + bt[...],
                   0.0)
  hv = jnp.maximum(jnp.dot(xv, wv[...], preferred_element_type=f32) + bv[...],
                   0.0)
  d = wt.shape[1]
  h1 = (jnp.dot(ht, wp1[0:d, :], preferred_element_type=f32) +
        jnp.dot(hv, wp1[d:2 * d, :], preferred_element_type=f32) + bp1[...])
  h1 = jnp.maximum(h1, 0.0)
  mu = jnp.mean(h1, axis=-1, keepdims=True)
  xc = h1 - mu
  var = jnp.mean(xc * xc, axis=-1, keepdims=True)
  hn = xc * lax.rsqrt(var + 1e-5) * g_ln[...] + be_ln[...]
  h_ref[...] = jnp.dot(hn, wp2[...], preferred_element_type=f32) + bp2[...]
  lut_ref[...] = jnp.dot(ht, wct[...], preferred_element_type=f32) + bct[...]
  luv_ref[...] = jnp.dot(hv, wcv[...], preferred_element_type=f32) + bcv[...]


def _prescale_body(h_ref, degs_ref, out_ref):
  dinv = lax.rsqrt(jnp.maximum(degs_ref[0, 0, :], 1.0))
  out_ref[...] = h_ref[...] * dinv[:, None]


def _dense2_body(agg0_ref, agg1_ref, degd_ref, wg, bg, wcc, bcc, lut_ref,
                 luv_ref, lf_ref, lc_ref):
  agg = agg0_ref[...] + agg1_ref[...]
  dinv = lax.rsqrt(jnp.maximum(degd_ref[0, 0, :], 1.0))
  hc = (jnp.dot(agg * dinv[:, None], wg[...], preferred_element_type=f32)
        + bg[...])
  lc = jnp.dot(hc, wcc[...], preferred_element_type=f32) + bcc[...]
  lc_ref[...] = lc
  lf_ref[...] = lc + lut_ref[...] + luv_ref[...]


def kernel(x_t, x_v, edge_index, W_t, b_t, W_v, b_v, W_p1, b_p1, g_ln, be_ln,
           W_p2, b_p2, W_g, b_g, W_cC, b_cC, W_ct, b_ct, W_cv, b_cv):
  N, D_T = x_t.shape
  D = W_t.shape[1]
  C = W_cC.shape[1]
  E = edge_index.shape[1]
  BN = 5000
  NB = N // BN

  NW = NC * NS
  n_chunks_pad = ((E // CH + NW * 8 - 1) // (NW * 8)) * NW * 8
  pad_n = n_chunks_pad * CH - E
  if pad_n:
    pad_idx = N + (jnp.arange(pad_n, dtype=jnp.int32) % 8)
    ei_pad = jnp.concatenate(
        [edge_index, jnp.stack([pad_idx, pad_idx])], axis=1)
  else:
    ei_pad = edge_index
  ei3_pad = ei_pad.reshape(2, n_chunks_pad, CH)
  ei_t = jnp.transpose(ei3_pad, (1, 0, 2))

  deg_s, deg_d = _degree_kernel(n_chunks_pad, N)(ei3_pad)
  deg_src = deg_s[:N].reshape(NB, 1, BN)
  deg_dst = deg_d[:N].reshape(NB, 1, BN)

  row = lambda v: v.reshape(1, -1)
  full = lambda shape: pl.BlockSpec(shape, lambda i: tuple(0 for _ in shape))

  h2, logits_Ut, logits_Uv = pl.pallas_call(
      _dense1_body,
      grid=(NB,),
      in_specs=[
          pl.BlockSpec((BN, D_T), lambda i: (i, 0)),
          pl.BlockSpec((BN, D_T), lambda i: (i, 0)),
          full((D_T, D)), full((1, D)),
          full((D_T, D)), full((1, D)),
          full((2 * D, D)), full((1, D)),
          full((1, D)), full((1, D)),
          full((D, D)), full((1, D)),
          full((D, C)), full((1, C)),
          full((D, C)), full((1, C)),
      ],
      out_specs=[
          pl.BlockSpec((BN, D), lambda i: (i, 0)),
          pl.BlockSpec((BN, C), lambda i: (i, 0)),
          pl.BlockSpec((BN, C), lambda i: (i, 0)),
      ],
      out_shape=[
          jax.ShapeDtypeStruct((N, D), f32),
          jax.ShapeDtypeStruct((N, C), f32),
          jax.ShapeDtypeStruct((N, C), f32),
      ],
  )(x_t, x_v, W_t, row(b_t), W_v, row(b_v), W_p1, row(b_p1), row(g_ln),
    row(be_ln), W_p2, row(b_p2), W_ct, row(b_ct), W_cv, row(b_cv))

  h_scaled = pl.pallas_call(
      _prescale_body,
      grid=(NB,),
      in_specs=[
          pl.BlockSpec((BN, D), lambda i: (i, 0)),
          pl.BlockSpec((1, 1, BN), lambda i: (i, 0, 0)),
      ],
      out_specs=pl.BlockSpec((BN, D), lambda i: (i, 0)),
      out_shape=jax.ShapeDtypeStruct((N + 8, D), f32),
  )(h2, deg_src)

  agg0, agg1 = _edge_agg_kernel(n_chunks_pad, N, D, split=0.575)(ei_t, h_scaled)

  logits_final, logits_C = pl.pallas_call(
      _dense2_body,
      grid=(NB,),
      in_specs=[
          pl.BlockSpec((BN, D), lambda i: (i, 0)),
          pl.BlockSpec((BN, D), lambda i: (i, 0)),
          pl.BlockSpec((1, 1, BN), lambda i: (i, 0, 0)),
          full((D, D)), full((1, D)),
          full((D, C)), full((1, C)),
          pl.BlockSpec((BN, C), lambda i: (i, 0)),
          pl.BlockSpec((BN, C), lambda i: (i, 0)),
      ],
      out_specs=[
          pl.BlockSpec((BN, C), lambda i: (i, 0)),
          pl.BlockSpec((BN, C), lambda i: (i, 0)),
      ],
      out_shape=[
          jax.ShapeDtypeStruct((N, C), f32),
          jax.ShapeDtypeStruct((N, C), f32),
      ],
  )(agg0, agg1, deg_dst, W_g, row(b_g), W_cC, row(b_cC),
    logits_Ut, logits_Uv)

  return (logits_final, logits_C, logits_Ut, logits_Uv)
